# baseline (speedup 1.0000x reference)
import functools

import jax
import jax.numpy as jnp
from jax import lax
from jax.experimental import pallas as pl
from jax.experimental.pallas import tpu as pltpu
from jax.experimental.pallas import tpu_sc as plsc

B = 16
N = 2048
K = 16384
L = 16
BLK = 128
K_HALF = K // 2


def _diag_body(t_ref, out_ref):
    blk = t_ref[0]
    ii = lax.broadcasted_iota(jnp.int32, (BLK, BLK), 0)
    jj = lax.broadcasted_iota(jnp.int32, (BLK, BLK), 1)
    out_ref[...] = jnp.sum(jnp.where(ii == jj, blk, 0.0), axis=1)


_diag_extract = pl.pallas_call(
    _diag_body,
    grid=(B, N // BLK),
    in_specs=[pl.BlockSpec((1, BLK, BLK), lambda b, k: (b, k, k))],
    out_specs=pl.BlockSpec((BLK,), lambda b, k: (b * (N // BLK) + k,)),
    out_shape=jax.ShapeDtypeStruct((B * N,), jnp.float32),
)


def _sc_body(diag_hbm, idx_hbm, out_hbm, diag_v, idx_v, out_v):
    c = lax.axis_index("c")
    s = lax.axis_index("s")

    pltpu.sync_copy(diag_hbm.at[pl.ds(s * N, N)], diag_v)
    base = c * K_HALF
    pltpu.sync_copy(idx_hbm.at[pl.ds(base, K_HALF)], idx_v)

    def gat(g, carry):
        for u in range(8):
            o = g * 8 * L + u * L
            iv = idx_v[pl.ds(o, L)]
            out_v[pl.ds(o, L)] = plsc.load_gather(diag_v, [iv])
        return carry
    lax.fori_loop(0, K_HALF // (8 * L), gat, 0)

    pltpu.sync_copy(out_v, out_hbm.at[s, pl.ds(base, K_HALF)])


_sc_lookup = functools.partial(
    pl.kernel,
    out_type=jax.ShapeDtypeStruct((B, K), jnp.float32),
    mesh=plsc.VectorSubcoreMesh(core_axis_name="c", subcore_axis_name="s"),
    compiler_params=pltpu.CompilerParams(needs_layout_passes=False),
    scratch_types=[
        pltpu.VMEM((N,), jnp.float32),
        pltpu.VMEM((K_HALF,), jnp.int32),
        pltpu.VMEM((K_HALF,), jnp.float32),
    ],
)(_sc_body)


def kernel(t, idx):
    diag = _diag_extract(t)
    return _sc_lookup(diag, idx.astype(jnp.int32))


# speedup vs baseline: 5.1735x; 5.1735x over previous
import functools

import jax
import jax.numpy as jnp
from jax import lax
from jax.experimental import pallas as pl
from jax.experimental.pallas import tpu as pltpu
from jax.experimental.pallas import tpu_sc as plsc

B = 16
N = 2048
K = 16384
L = 16
BLK = 128
NSTREAM = 4
K_HALF = K // 2


def _diag_body(*refs):
    t_refs, out_refs = refs[:NSTREAM], refs[NSTREAM:]
    ii = lax.broadcasted_iota(jnp.int32, (BLK, BLK), 0)
    jj = lax.broadcasted_iota(jnp.int32, (BLK, BLK), 1)
    eq = ii == jj
    for t_ref, out_ref in zip(t_refs, out_refs):
        blk = t_ref[...]
        out_ref[...] = jnp.sum(jnp.where(eq[None], blk, 0.0), axis=2)


def _in_spec(j):
    return pl.BlockSpec((B, BLK, BLK),
                        lambda k, j=j: (0, NSTREAM * j + k, NSTREAM * j + k))


_diag_extract = pl.pallas_call(
    _diag_body,
    grid=(N // BLK // NSTREAM,),
    in_specs=[_in_spec(j) for j in range(NSTREAM)],
    out_specs=[pl.BlockSpec((B, BLK), lambda k: (0, k))] * NSTREAM,
    out_shape=[jax.ShapeDtypeStruct((B, N // NSTREAM), jnp.float32)] * NSTREAM,
)


def _sc_body(diag_hbm, idx_hbm, out_hbm, diag_v, idx_v, out_v):
    c = lax.axis_index("c")
    s = lax.axis_index("s")

    pltpu.sync_copy(diag_hbm.at[pl.ds(s * N, N)], diag_v)
    base = c * K_HALF
    pltpu.sync_copy(idx_hbm.at[pl.ds(base, K_HALF)], idx_v)

    def gat(g, carry):
        for u in range(8):
            o = g * 8 * L + u * L
            iv = idx_v[pl.ds(o, L)]
            out_v[pl.ds(o, L)] = plsc.load_gather(diag_v, [iv])
        return carry
    lax.fori_loop(0, K_HALF // (8 * L), gat, 0)

    pltpu.sync_copy(out_v, out_hbm.at[s, pl.ds(base, K_HALF)])


_sc_lookup = functools.partial(
    pl.kernel,
    out_type=jax.ShapeDtypeStruct((B, K), jnp.float32),
    mesh=plsc.VectorSubcoreMesh(core_axis_name="c", subcore_axis_name="s"),
    compiler_params=pltpu.CompilerParams(needs_layout_passes=False),
    scratch_types=[
        pltpu.VMEM((N,), jnp.float32),
        pltpu.VMEM((K_HALF,), jnp.int32),
        pltpu.VMEM((K_HALF,), jnp.float32),
    ],
)(_sc_body)


def kernel(t, idx):
    quarters = _diag_extract(t, t, t, t)
    diag = jnp.concatenate(quarters, axis=1).reshape(B * N)
    return _sc_lookup(diag, idx.astype(jnp.int32))
